# baseline (speedup 1.0000x reference)
import functools

import jax
import jax.numpy as jnp
from jax import lax
from jax.experimental import pallas as pl
from jax.experimental.pallas import tpu as pltpu
from jax.experimental.pallas import tpu_sc as plsc

NR_GROUPS = 16
OUT_DIM = 128
KW = 16
BS, CH, T = 32, 64, 128
T_OUT = T - KW + 1
N = BS * CH

_NC = 2
_NS = 16
_NW = _NC * _NS
_ROWS_PER_TILE = N // _NW
_SEG_ROWS = BS * NR_GROUPS
_SEG_PER_TILE = _SEG_ROWS // _NW


def _seg_sum_body(x_hbm, gid_hbm, out_hbm, cnt_hbm, idx_v, rows_v, acc_v, cnt_v,
                  sem_i, sem_r):
    wid = lax.axis_index("s") * _NC + lax.axis_index("c")
    row_base = wid * _ROWS_PER_TILE
    cp_idx = pltpu.async_copy(gid_hbm.at[pl.ds(row_base, _ROWS_PER_TILE)],
                              idx_v.at[pl.ds(0, _ROWS_PER_TILE)], sem_i)
    cp_rows = pltpu.async_copy(x_hbm.at[pl.ds(row_base * T, _ROWS_PER_TILE * T)],
                               rows_v, sem_r)
    zv = jnp.zeros((16,), jnp.float32)
    for j in range(NR_GROUPS * T // 16):
        acc_v[pl.ds(j * 16, 16)] = zv
    cp_idx.wait()
    cp_rows.wait()
    @plsc.parallel_loop(0, _ROWS_PER_TILE, 1, unroll=16)
    def _acc_loop(ch):
        gv = idx_v[pl.ds(ch, 16)]
        base = gv[0] * T
        for j in range(T // 16):
            plsc.addupdate(acc_v.at[pl.ds(base + 16 * j, 16)],
                           rows_v[pl.ds(ch * T + 16 * j, 16)])
    cnt = jnp.zeros((16,), jnp.float32)
    for cblk in range(_ROWS_PER_TILE // 16):
        gv = idx_v[pl.ds(cblk * 16, 16)]
        cnt = cnt + jnp.where(gv == 0, 1.0, 0.0).astype(jnp.float32)
    cnt_v[...] = cnt
    pltpu.sync_copy(cnt_v, cnt_hbm.at[pl.ds(wid * 16, 16)])
    pltpu.sync_copy(acc_v, out_hbm.at[pl.ds(wid * _SEG_PER_TILE * T,
                                            _SEG_PER_TILE * T)])


@functools.cache
def _get_seg_sum():
    return pl.kernel(
        _seg_sum_body,
        mesh=plsc.VectorSubcoreMesh(core_axis_name="c", subcore_axis_name="s"),
        out_type=(jax.ShapeDtypeStruct((_SEG_ROWS * T,), jnp.float32),
                  jax.ShapeDtypeStruct((_NW * 16,), jnp.float32)),
        scratch_types=[
            pltpu.VMEM((_ROWS_PER_TILE + 16,), jnp.int32),
            pltpu.VMEM((_ROWS_PER_TILE * T,), jnp.float32),
            pltpu.VMEM((NR_GROUPS * T,), jnp.float32),
            pltpu.VMEM((16,), jnp.float32),
            pltpu.SemaphoreType.DMA,
            pltpu.SemaphoreType.DMA,
        ],
    )


def _conv_body(xsum_ref, cnt_ref, wmt_ref, b0_ref, out_ref):
    xs = xsum_ref[...]
    wmt = wmt_ref[...]
    b0 = b0_ref[...]
    cnt = jnp.sum(cnt_ref[...], axis=1, keepdims=True)
    for b in range(BS):
        xb = xs[b * NR_GROUPS:(b + 1) * NR_GROUPS, :]
        winT = jnp.concatenate(
            [xb[:, k:k + T_OUT] for k in range(KW)], axis=0)
        ob_t = lax.dot_general(winT, wmt, (((0,), (0,)), ((), ())),
                               preferred_element_type=jnp.float32)
        out_ref[:, b * OUT_DIM:(b + 1) * OUT_DIM] = (
            ob_t + cnt[b:b + 1, 0:1] * b0)


_conv = pl.pallas_call(
    _conv_body,
    out_shape=jax.ShapeDtypeStruct((T_OUT, BS * OUT_DIM), jnp.float32),
)


def kernel(x, group_idxs, W, bias0):
    x_flat = x.reshape(N * T)
    xsum_flat, cnts = _get_seg_sum()(x_flat, group_idxs.reshape(N))
    xsum = xsum_flat.reshape(_SEG_ROWS, T)
    wmt = jnp.transpose(W, (2, 0, 1)).reshape(NR_GROUPS * KW, OUT_DIM)
    b0r = bias0.reshape(1, OUT_DIM)
    out3 = _conv(xsum, cnts.reshape(BS, 16), wmt, b0r)
    return jnp.transpose(out3.reshape(T_OUT, BS, OUT_DIM), (1, 2, 0))


# speedup vs baseline: 1.0445x; 1.0445x over previous
import functools

import jax
import jax.numpy as jnp
from jax import lax
from jax.experimental import pallas as pl
from jax.experimental.pallas import tpu as pltpu
from jax.experimental.pallas import tpu_sc as plsc

NR_GROUPS = 16
OUT_DIM = 128
KW = 16
BS, CH, T = 32, 64, 128
T_OUT = T - KW + 1
N = BS * CH

_NC = 2
_NS = 16
_NW = _NC * _NS
_ROWS_PER_TILE = N // _NW
_SEG_ROWS = BS * NR_GROUPS
_SEG_PER_TILE = _SEG_ROWS // _NW


def _seg_sum_body(x_hbm, gid_hbm, out_hbm, idx_v, rows_v, acc_v, sem_i, sem_r):
    wid = lax.axis_index("s") * _NC + lax.axis_index("c")
    row_base = wid * _ROWS_PER_TILE
    cp_idx = pltpu.async_copy(gid_hbm.at[pl.ds(row_base, _ROWS_PER_TILE)],
                              idx_v.at[pl.ds(0, _ROWS_PER_TILE)], sem_i)
    cp_rows = pltpu.async_copy(x_hbm.at[pl.ds(row_base * T, _ROWS_PER_TILE * T)],
                               rows_v, sem_r)
    zv = jnp.zeros((16,), jnp.float32)
    for j in range(NR_GROUPS * T // 16):
        acc_v[pl.ds(j * 16, 16)] = zv
    cp_idx.wait()
    cp_rows.wait()
    @plsc.parallel_loop(0, _ROWS_PER_TILE, 1, unroll=16)
    def _acc_loop(ch):
        gv = idx_v[pl.ds(ch, 16)]
        base = gv[0] * T
        for j in range(T // 16):
            plsc.addupdate(acc_v.at[pl.ds(base + 16 * j, 16)],
                           rows_v[pl.ds(ch * T + 16 * j, 16)])
    pltpu.sync_copy(acc_v, out_hbm.at[pl.ds(wid * _SEG_PER_TILE * T,
                                            _SEG_PER_TILE * T)])


@functools.cache
def _get_seg_sum():
    return pl.kernel(
        _seg_sum_body,
        mesh=plsc.VectorSubcoreMesh(core_axis_name="c", subcore_axis_name="s"),
        out_type=jax.ShapeDtypeStruct((_SEG_ROWS * T,), jnp.float32),
        scratch_types=[
            pltpu.VMEM((_ROWS_PER_TILE + 16,), jnp.int32),
            pltpu.VMEM((_ROWS_PER_TILE * T,), jnp.float32),
            pltpu.VMEM((NR_GROUPS * T,), jnp.float32),
            pltpu.SemaphoreType.DMA,
            pltpu.SemaphoreType.DMA,
        ],
    )


def _conv_body(xsum_ref, gid_ref, wmt_ref, b0_ref, out_ref):
    xs = xsum_ref[...]
    wmt = wmt_ref[...]
    b0 = b0_ref[...]
    gid = gid_ref[...]
    cnt = jnp.sum((gid == 0).astype(jnp.float32), axis=1, keepdims=True)
    for b in range(BS):
        xb = xs[b * NR_GROUPS:(b + 1) * NR_GROUPS, :]
        winT = jnp.concatenate(
            [xb[:, k:k + T_OUT] for k in range(KW)], axis=0)
        ob_t = lax.dot_general(winT, wmt, (((0,), (0,)), ((), ())),
                               preferred_element_type=jnp.float32)
        out_ref[:, b * OUT_DIM:(b + 1) * OUT_DIM] = (
            ob_t + cnt[b:b + 1, 0:1] * b0)


_conv = pl.pallas_call(
    _conv_body,
    out_shape=jax.ShapeDtypeStruct((T_OUT, BS * OUT_DIM), jnp.float32),
)


def kernel(x, group_idxs, W, bias0):
    x_flat = x.reshape(N * T)
    xsum = _get_seg_sum()(x_flat, group_idxs.reshape(N)).reshape(_SEG_ROWS, T)
    wmt = jnp.transpose(W, (2, 0, 1)).reshape(NR_GROUPS * KW, OUT_DIM)
    b0r = bias0.reshape(1, OUT_DIM)
    out3 = _conv(xsum, group_idxs, wmt, b0r)
    return jnp.transpose(out3.reshape(T_OUT, BS, OUT_DIM), (1, 2, 0))
